# baseline (speedup 1.0000x reference)
import functools

import jax
import jax.numpy as jnp
from jax import lax
from jax.experimental import pallas as pl
from jax.experimental.pallas import tpu as pltpu
from jax.experimental.pallas import tpu_sc as plsc

_B, _S, _D = 4096, 200, 128
_N = _B * _S
_EPS = 1e-12
_V_TYPE, _V_DPE = 14, 25

_NC, _NS = 2, 16
_NW = _NC * _NS
_RPW = _N // _NW
_CHUNK = 128
_NCHUNK = _RPW // _CHUNK


def _sc_gather_sum(ids, ct, w_in, w_td):
    mesh = plsc.VectorSubcoreMesh(core_axis_name="c", subcore_axis_name="s")

    @functools.partial(
        pl.kernel,
        out_type=jax.ShapeDtypeStruct((_N, _D), jnp.float32),
        mesh=mesh,
        scratch_types=[
            pltpu.VMEM((_CHUNK,), jnp.int32),
            pltpu.VMEM((_CHUNK,), jnp.int32),
            pltpu.VMEM((_CHUNK, _D), jnp.float32),
            pltpu.VMEM((_V_TYPE * _V_DPE, _D), jnp.float32),
            pltpu.SemaphoreType.DMA,
        ],
    )
    def k(ids_hbm, ct_hbm, win_hbm, wtd_hbm, out_hbm, idx_v, ct_v, a_v, wtd_v, sem):
        wid = lax.axis_index("s") * _NC + lax.axis_index("c")
        base = wid * _RPW
        pltpu.sync_copy(wtd_hbm, wtd_v)

        def chunk_body(c, carry):
            off = base + c * _CHUNK
            pltpu.sync_copy(ids_hbm.at[pl.ds(off, _CHUNK)], idx_v)
            pltpu.sync_copy(ct_hbm.at[pl.ds(off, _CHUNK)], ct_v)
            pltpu.async_copy(win_hbm.at[idx_v], a_v, sem).wait()

            def add_body(g, c2):
                ct16 = ct_v[pl.ds(g * 16, 16)]
                for i in range(16):
                    row = ct16[i]
                    for j in range(_D // 16):
                        sl = pl.ds(j * 16, 16)
                        a_v[g * 16 + i, sl] += wtd_v[row, sl]
                return c2

            lax.fori_loop(0, _CHUNK // 16, add_body, 0)
            pltpu.sync_copy(a_v, out_hbm.at[pl.ds(off, _CHUNK)])
            return carry

        lax.fori_loop(0, _NCHUNK, chunk_body, 0)

    return k(ids, ct, w_in, w_td)


_RBLK = 1024


def _tc_layernorm(x, gamma, beta):
    def body(x_ref, g_ref, b_ref, o_ref):
        xv = x_ref[...]
        mean = jnp.mean(xv, axis=1, keepdims=True)
        xc = xv - mean
        var = jnp.mean(xc * xc, axis=1, keepdims=True)
        o_ref[...] = xc * lax.rsqrt(var + _EPS) * g_ref[...] + b_ref[...]

    return pl.pallas_call(
        body,
        grid=(_N // _RBLK,),
        in_specs=[
            pl.BlockSpec((_RBLK, _D), lambda i: (i, 0)),
            pl.BlockSpec((1, _D), lambda i: (0, 0)),
            pl.BlockSpec((1, _D), lambda i: (0, 0)),
        ],
        out_specs=pl.BlockSpec((_RBLK, _D), lambda i: (i, 0)),
        out_shape=jax.ShapeDtypeStruct((_N, _D), jnp.float32),
    )(x, gamma.reshape(1, _D), beta.reshape(1, _D))


def kernel(input_ids, type_ids, dpe_ids, W_input, W_type, W_dpe, gamma, beta):
    ids = input_ids.reshape(_N).astype(jnp.int32)
    ct = (type_ids.reshape(_N).astype(jnp.int32) * _V_DPE
          + dpe_ids.reshape(_N).astype(jnp.int32))
    w_td = (W_type[:, None, :] + W_dpe[None, :, :]).reshape(_V_TYPE * _V_DPE, _D)
    s = _sc_gather_sum(ids, ct, W_input, w_td)
    y = _tc_layernorm(s, gamma, beta)
    return y.reshape(_B, _S, _D)


# speedup vs baseline: 1.3332x; 1.3332x over previous
import functools

import jax
import jax.numpy as jnp
from jax import lax
from jax.experimental import pallas as pl
from jax.experimental.pallas import tpu as pltpu
from jax.experimental.pallas import tpu_sc as plsc

_B, _S, _D = 4096, 200, 128
_N = _B * _S
_EPS = 1e-12
_V_TYPE, _V_DPE = 14, 25

_NC, _NS = 2, 16
_NW = _NC * _NS
_RPW = _N // _NW
_CHUNK = 128
_NCHUNK = _RPW // _CHUNK


def _sc_gather_sum(ids, ct, w_in, w_td):
    mesh = plsc.VectorSubcoreMesh(core_axis_name="c", subcore_axis_name="s")

    @functools.partial(
        pl.kernel,
        out_type=jax.ShapeDtypeStruct((_N, _D), jnp.float32),
        mesh=mesh,
        scratch_types=[
            pltpu.VMEM((_CHUNK,), jnp.int32),
            pltpu.VMEM((_CHUNK,), jnp.int32),
            pltpu.VMEM((_CHUNK, _D), jnp.float32),
            pltpu.SemaphoreType.DMA,
        ],
    )
    def k(ids_hbm, ct_hbm, win_hbm, wtd_hbm, out_hbm, idx_v, ct_v, a_v, sem):
        wid = lax.axis_index("s") * _NC + lax.axis_index("c")
        base = wid * _RPW

        def chunk_body(c, carry):
            off = base + c * _CHUNK
            pltpu.sync_copy(ids_hbm.at[pl.ds(off, _CHUNK)], idx_v)
            pltpu.sync_copy(ct_hbm.at[pl.ds(off, _CHUNK)], ct_v)
            pltpu.async_copy(win_hbm.at[idx_v], a_v, sem).wait()
            pltpu.sync_copy(wtd_hbm.at[ct_v], a_v, add=True)
            pltpu.sync_copy(a_v, out_hbm.at[pl.ds(off, _CHUNK)])
            return carry

        lax.fori_loop(0, _NCHUNK, chunk_body, 0)

    return k(ids, ct, w_in, w_td)


_RBLK = 1024


def _tc_layernorm(x, gamma, beta):
    def body(x_ref, g_ref, b_ref, o_ref):
        xv = x_ref[...]
        mean = jnp.mean(xv, axis=1, keepdims=True)
        xc = xv - mean
        var = jnp.mean(xc * xc, axis=1, keepdims=True)
        o_ref[...] = xc * lax.rsqrt(var + _EPS) * g_ref[...] + b_ref[...]

    return pl.pallas_call(
        body,
        grid=(_N // _RBLK,),
        in_specs=[
            pl.BlockSpec((_RBLK, _D), lambda i: (i, 0)),
            pl.BlockSpec((1, _D), lambda i: (0, 0)),
            pl.BlockSpec((1, _D), lambda i: (0, 0)),
        ],
        out_specs=pl.BlockSpec((_RBLK, _D), lambda i: (i, 0)),
        out_shape=jax.ShapeDtypeStruct((_N, _D), jnp.float32),
    )(x, gamma.reshape(1, _D), beta.reshape(1, _D))


def kernel(input_ids, type_ids, dpe_ids, W_input, W_type, W_dpe, gamma, beta):
    ids = input_ids.reshape(_N).astype(jnp.int32)
    ct = (type_ids.reshape(_N).astype(jnp.int32) * _V_DPE
          + dpe_ids.reshape(_N).astype(jnp.int32))
    w_td = (W_type[:, None, :] + W_dpe[None, :, :]).reshape(_V_TYPE * _V_DPE, _D)
    s = _sc_gather_sum(ids, ct, W_input, w_td)
    y = _tc_layernorm(s, gamma, beta)
    return y.reshape(_B, _S, _D)


# speedup vs baseline: 1.5413x; 1.1560x over previous
import functools

import jax
import jax.numpy as jnp
from jax import lax
from jax.experimental import pallas as pl
from jax.experimental.pallas import tpu as pltpu
from jax.experimental.pallas import tpu_sc as plsc

_B, _S, _D = 4096, 200, 128
_N = _B * _S
_EPS = 1e-12
_V_TYPE, _V_DPE = 14, 25

_NC, _NS = 2, 16
_NW = _NC * _NS
_RPW = _N // _NW
_CHUNK = 128
_NCHUNK = _RPW // _CHUNK


def _sc_gather_sum(ids, ct, w_in, w_td):
    mesh = plsc.VectorSubcoreMesh(core_axis_name="c", subcore_axis_name="s")

    nbuf = 4

    @functools.partial(
        pl.kernel,
        out_type=jax.ShapeDtypeStruct((_N, _D), jnp.float32),
        mesh=mesh,
        scratch_types=[
            pltpu.VMEM((_RPW,), jnp.int32),
            pltpu.VMEM((_RPW,), jnp.int32),
        ]
        + [pltpu.VMEM((_CHUNK, _D), jnp.float32) for _ in range(nbuf)]
        + [pltpu.SemaphoreType.DMA for _ in range(3 * nbuf)],
    )
    def k(ids_hbm, ct_hbm, win_hbm, wtd_hbm, out_hbm, idx_v, ct_v, *rest):
        bufs = rest[:nbuf]
        sg = rest[nbuf:2 * nbuf]
        sa = rest[2 * nbuf:3 * nbuf]
        so = rest[3 * nbuf:4 * nbuf]
        wid = lax.axis_index("s") * _NC + lax.axis_index("c")
        base = wid * _RPW

        pltpu.sync_copy(ids_hbm.at[pl.ds(base, _RPW)], idx_v)
        pltpu.sync_copy(ct_hbm.at[pl.ds(base, _RPW)], ct_v)

        def gather_desc(c, p):
            return pltpu.make_async_copy(
                win_hbm.at[idx_v.at[pl.ds(c * _CHUNK, _CHUNK)]], bufs[p], sg[p])

        def add_desc(c, p):
            return pltpu.make_async_copy(
                wtd_hbm.at[ct_v.at[pl.ds(c * _CHUNK, _CHUNK)]], bufs[p], sa[p])

        def out_desc(c, p):
            return pltpu.make_async_copy(
                bufs[p], out_hbm.at[pl.ds(base + c * _CHUNK, _CHUNK)], so[p])

        def issue_add(c, p):
            pltpu.async_copy(
                wtd_hbm.at[ct_v.at[pl.ds(c * _CHUNK, _CHUNK)]], bufs[p], sa[p],
                add=True)

        for p in range(nbuf - 1):
            gather_desc(p, p).start()

        def body(g, carry):
            for p in range(nbuf):
                c = g * nbuf + p
                pn = (p + nbuf - 1) % nbuf
                gather_desc(c, p).wait()
                issue_add(c, p)
                if p == 0:
                    @pl.when(g > 0)
                    def _():
                        out_desc(c - 1, pn).wait()
                else:
                    out_desc(c - 1, pn).wait()
                if p == 0:
                    gather_desc(c + nbuf - 1, pn).start()
                else:
                    @pl.when(g < _NCHUNK // nbuf - 1)
                    def _():
                        gather_desc(c + nbuf - 1, pn).start()
                add_desc(c, p).wait()
                out_desc(c, p).start()
            return carry

        lax.fori_loop(0, _NCHUNK // nbuf, body, 0)
        out_desc(_NCHUNK - 1, nbuf - 1).wait()

    return k(ids, ct, w_in, w_td)


_RBLK = 1024


def _tc_layernorm(x, gamma, beta):
    def body(x_ref, g_ref, b_ref, o_ref):
        xv = x_ref[...]
        mean = jnp.mean(xv, axis=1, keepdims=True)
        xc = xv - mean
        var = jnp.mean(xc * xc, axis=1, keepdims=True)
        o_ref[...] = xc * lax.rsqrt(var + _EPS) * g_ref[...] + b_ref[...]

    return pl.pallas_call(
        body,
        grid=(_N // _RBLK,),
        in_specs=[
            pl.BlockSpec((_RBLK, _D), lambda i: (i, 0)),
            pl.BlockSpec((1, _D), lambda i: (0, 0)),
            pl.BlockSpec((1, _D), lambda i: (0, 0)),
        ],
        out_specs=pl.BlockSpec((_RBLK, _D), lambda i: (i, 0)),
        out_shape=jax.ShapeDtypeStruct((_N, _D), jnp.float32),
    )(x, gamma.reshape(1, _D), beta.reshape(1, _D))


def kernel(input_ids, type_ids, dpe_ids, W_input, W_type, W_dpe, gamma, beta):
    ids = input_ids.reshape(_N).astype(jnp.int32)
    ct = (type_ids.reshape(_N).astype(jnp.int32) * _V_DPE
          + dpe_ids.reshape(_N).astype(jnp.int32))
    w_td = (W_type[:, None, :] + W_dpe[None, :, :]).reshape(_V_TYPE * _V_DPE, _D)
    s = _sc_gather_sum(ids, ct, W_input, w_td)
    y = _tc_layernorm(s, gamma, beta)
    return y.reshape(_B, _S, _D)


# speedup vs baseline: 1.8789x; 1.2191x over previous
import functools

import jax
import jax.numpy as jnp
from jax import lax
from jax.experimental import pallas as pl
from jax.experimental.pallas import tpu as pltpu
from jax.experimental.pallas import tpu_sc as plsc

_B, _S, _D = 4096, 200, 128
_N = _B * _S
_EPS = 1e-12
_V_TYPE, _V_DPE = 14, 25

_NC, _NS = 2, 16
_NW = _NC * _NS
_NSLAB = 4
_NSL = _N // _NSLAB
_RPW = _NSL // _NW
_CHUNK = 128
_NCHUNK = _RPW // _CHUNK
_NBUF = 4


def _sc_gather_sum(ids, ct, w_in, w_td):
    mesh = plsc.VectorSubcoreMesh(core_axis_name="c", subcore_axis_name="s")

    @functools.partial(
        pl.kernel,
        out_type=jax.ShapeDtypeStruct((_NSL, _D), jnp.float32),
        mesh=mesh,
        scratch_types=[
            pltpu.VMEM((_RPW,), jnp.int32),
            pltpu.VMEM((_RPW,), jnp.int32),
        ]
        + [pltpu.VMEM((_CHUNK, _D), jnp.float32) for _ in range(_NBUF)]
        + [pltpu.SemaphoreType.DMA for _ in range(3 * _NBUF)],
    )
    def k(ids_hbm, ct_hbm, win_hbm, wtd_hbm, out_hbm, idx_v, ct_v, *rest):
        bufs = rest[:_NBUF]
        sg = rest[_NBUF:2 * _NBUF]
        sa = rest[2 * _NBUF:3 * _NBUF]
        so = rest[3 * _NBUF:4 * _NBUF]
        wid = lax.axis_index("s") * _NC + lax.axis_index("c")
        base = wid * _RPW

        pltpu.sync_copy(ids_hbm.at[pl.ds(base, _RPW)], idx_v)
        pltpu.sync_copy(ct_hbm.at[pl.ds(base, _RPW)], ct_v)

        def gather_desc(c, p):
            return pltpu.make_async_copy(
                win_hbm.at[idx_v.at[pl.ds(c * _CHUNK, _CHUNK)]], bufs[p], sg[p])

        def add_desc(c, p):
            return pltpu.make_async_copy(
                wtd_hbm.at[ct_v.at[pl.ds(c * _CHUNK, _CHUNK)]], bufs[p], sa[p])

        def out_desc(c, p):
            return pltpu.make_async_copy(
                bufs[p], out_hbm.at[pl.ds(base + c * _CHUNK, _CHUNK)], so[p])

        def issue_add(c, p):
            pltpu.async_copy(
                wtd_hbm.at[ct_v.at[pl.ds(c * _CHUNK, _CHUNK)]], bufs[p], sa[p],
                add=True)

        for p in range(_NBUF - 1):
            gather_desc(p, p).start()

        for c in range(_NCHUNK):
            p = c % _NBUF
            pn = (p + _NBUF - 1) % _NBUF
            gather_desc(c, p).wait()
            issue_add(c, p)
            if c >= 1:
                out_desc(c - 1, pn).wait()
            if c + _NBUF - 1 < _NCHUNK:
                gather_desc(c + _NBUF - 1, pn).start()
            add_desc(c, p).wait()
            out_desc(c, p).start()
        out_desc(_NCHUNK - 1, (_NCHUNK - 1) % _NBUF).wait()

    return k(ids, ct, w_in, w_td)


_RBLK = 1024
_SBLKS = _NSL // _RBLK


def _tc_layernorm_slab(x_slab, gamma, beta, y_prev, slab):
    def body(*refs):
        x_ref, g_ref, b_ref = refs[0], refs[1], refs[2]
        o_ref = refs[-1]
        xv = x_ref[...]
        mean = jnp.mean(xv, axis=1, keepdims=True)
        xc = xv - mean
        var = jnp.mean(xc * xc, axis=1, keepdims=True)
        o_ref[...] = xc * lax.rsqrt(var + _EPS) * g_ref[...] + b_ref[...]

    in_specs = [
        pl.BlockSpec((_RBLK, _D), lambda i: (i, 0)),
        pl.BlockSpec((1, _D), lambda i: (0, 0)),
        pl.BlockSpec((1, _D), lambda i: (0, 0)),
    ]
    args = [x_slab, gamma.reshape(1, _D), beta.reshape(1, _D)]
    aliases = {}
    if y_prev is not None:
        in_specs.append(pl.BlockSpec(memory_space=pl.ANY))
        args.append(y_prev)
        aliases = {3: 0}
    return pl.pallas_call(
        body,
        grid=(_SBLKS,),
        in_specs=in_specs,
        out_specs=pl.BlockSpec((_RBLK, _D), lambda i, _s=slab: (i + _s * _SBLKS, 0)),
        out_shape=jax.ShapeDtypeStruct((_N, _D), jnp.float32),
        input_output_aliases=aliases,
    )(*args)


def kernel(input_ids, type_ids, dpe_ids, W_input, W_type, W_dpe, gamma, beta):
    ids = input_ids.reshape(_N).astype(jnp.int32)
    ct = (type_ids.reshape(_N).astype(jnp.int32) * _V_DPE
          + dpe_ids.reshape(_N).astype(jnp.int32))
    w_td = (W_type[:, None, :] + W_dpe[None, :, :]).reshape(_V_TYPE * _V_DPE, _D)
    y = None
    for s in range(_NSLAB):
        sum_s = _sc_gather_sum(ids[s * _NSL:(s + 1) * _NSL],
                               ct[s * _NSL:(s + 1) * _NSL], W_input, w_td)
        y = _tc_layernorm_slab(sum_s, gamma, beta, y, s)
    return y.reshape(_B, _S, _D)
